# baseline (speedup 1.0000x reference)
import functools

import jax
import jax.numpy as jnp
from jax import lax
from jax.experimental import pallas as pl
from jax.experimental.pallas import tpu as pltpu
from jax.experimental.pallas import tpu_sc as plsc

NC = 2
NS = 16
L = 16
NW = NC * NS

B = 16384
D = 64
W = 128
V = 1000000
CHUNK = 128
BPW = B // NW
KCH = BPW // CHUNK
JG = D // L

NFULL = V // W
TAIL = V - NFULL * W
ROUNDS = NFULL // NW
REM = NFULL - ROUNDS * NW

_mesh = plsc.VectorSubcoreMesh(core_axis_name="c", subcore_axis_name="s")


@functools.partial(
    pl.kernel,
    mesh=_mesh,
    out_type=jax.ShapeDtypeStruct((V, W), jnp.float32),
    compiler_params=pltpu.CompilerParams(use_tc_tiling_on_sc=True,
                                         needs_layout_passes=False),
    scratch_types=[
        pltpu.VMEM((2, D, W), jnp.float32),
        pltpu.VMEM((2, W, W), jnp.float32),
        pltpu.VMEM((TAIL, D), jnp.float32),
        pltpu.SemaphoreType.DMA,
        pltpu.SemaphoreType.DMA,
    ],
)
def _widen_sc(tab_t_hbm, tail_hbm, out_hbm, slab_v, obuf_v, tl_v, isem, osem):
    c = lax.axis_index("c")
    s = lax.axis_index("s")
    wid = c * NS + s

    f_idx = [lax.iota(jnp.int32, L) + jj * L for jj in range(JG)]

    def block_of(i):
        return wid + i * NW

    def fire_in(i, buf):
        j = block_of(i)
        return pltpu.async_copy(
            tab_t_hbm.at[:, pl.ds(pl.multiple_of(j * W, W), W)],
            slab_v.at[buf], isem)

    def transpose(buf):
        def col_step(cc, carry):
            ci = jnp.full((L,), cc, jnp.int32)
            for jj in range(JG):
                v = plsc.load_gather(slab_v.at[buf], [f_idx[jj], ci])
                plsc.store_scatter(obuf_v.at[buf], [ci, f_idx[jj]], v)
            return carry
        lax.fori_loop(0, W, col_step, 0, unroll=4)

    def fire_out(i, buf):
        j = block_of(i)
        return pltpu.async_copy(
            obuf_v.at[buf],
            out_hbm.at[pl.ds(pl.multiple_of(j * W, W), W)], osem)

    def wait_in(buf):
        pltpu.make_async_copy(
            tab_t_hbm.at[:, pl.ds(0, W)], slab_v.at[buf], isem).wait()

    def wait_out(buf):
        pltpu.make_async_copy(
            obuf_v.at[buf], out_hbm.at[pl.ds(0, W)], osem).wait()

    def fire_next(i):
        jn = jnp.where(i + 1 < ROUNDS,
                       wid + (i + 1) * NW,
                       ROUNDS * NW + jnp.minimum(wid, REM - 1))
        return pltpu.async_copy(
            tab_t_hbm.at[:, pl.ds(pl.multiple_of(jn * W, W), W)],
            slab_v.at[(i + 1) % 2], isem)

    fire_in(0, 0)
    def pair_step(g, carry):
        for b in range(2):
            i = g * 2 + b
            wait_in(b)
            fire_next(i)
            @pl.when(i >= 2)
            def _():
                wait_out(b)
            transpose(b)
            fire_out(i, b)
        return carry
    lax.fori_loop(0, ROUNDS // 2, pair_step, 0)
    wait_out(0)
    wait_out(1)
    wait_in(ROUNDS % 2)

    @pl.when(wid < REM)
    def _():
        buf = ROUNDS % 2
        jlast = ROUNDS * NW + wid
        transpose(buf)
        pltpu.async_copy(
            obuf_v.at[buf],
            out_hbm.at[pl.ds(pl.multiple_of(jlast * W, W), W)], osem).wait()

    @pl.when(wid == REM)
    def _():
        buf = ROUNDS % 2
        pltpu.sync_copy(tail_hbm, tl_v)

        def tail_step(cc, carry):
            ci = jnp.full((L,), cc, jnp.int32)
            for jj in range(JG):
                v = tl_v[cc, pl.ds(jj * L, L)]
                plsc.store_scatter(obuf_v.at[buf], [ci, f_idx[jj]], v)
            return carry
        lax.fori_loop(0, TAIL, tail_step, 0, unroll=4)
        pltpu.async_copy(
            obuf_v.at[buf, pl.ds(0, TAIL)],
            out_hbm.at[pl.ds(NFULL * W, TAIL)], osem).wait()


@functools.partial(
    pl.kernel,
    mesh=_mesh,
    out_type=jax.ShapeDtypeStruct((NW, L), jnp.float32),
    compiler_params=pltpu.CompilerParams(use_tc_tiling_on_sc=True),
    scratch_types=[
        pltpu.VMEM((2 * KCH, CHUNK), jnp.int32),
        pltpu.VMEM((KCH, CHUNK, W), jnp.float32),
        pltpu.VMEM((KCH, CHUNK // 2, W), jnp.float32),
        pltpu.VMEM((L,), jnp.float32),
        pltpu.SemaphoreType.DMA,
        pltpu.SemaphoreType.DMA,
    ],
)
def _center_loss_sc(z_hbm, lab_hbm, tab_hbm, out_hbm,
                    idx_v, rows_v, z_v, acc_v, sem, zsem):
    c = lax.axis_index("c")
    s = lax.axis_index("s")
    wid = c * NS + s

    pltpu.sync_copy(lab_hbm.at[wid], idx_v)
    zcp = pltpu.async_copy(z_hbm.at[wid], z_v, zsem)
    gathers = [
        pltpu.async_copy(tab_hbm.at[idx_v.at[k]], rows_v.at[k], sem)
        for k in range(KCH)
    ]
    zcp.wait()
    for cp in gathers:
        cp.wait()

    accs = (jnp.zeros((L,), jnp.float32),) * JG
    for k in range(KCH):
        def row_step(i, a, k=k):
            out = list(a)
            for p in range(2):
                for j in range(JG):
                    zv = z_v[k, i, pl.ds(p * D + j * L, L)]
                    cv = rows_v[k, 2 * i + p, pl.ds(j * L, L)]
                    d = zv - cv
                    out[j] = out[j] + d * d
            return tuple(out)
        accs = lax.fori_loop(0, CHUNK // 2, row_step, accs)

    tot = accs[0]
    for j in range(1, JG):
        tot = tot + accs[j]
    acc_v[...] = tot * (1.0 / B)
    pltpu.sync_copy(acc_v, out_hbm.at[wid])


def kernel(z, labels, centers):
    tab = _widen_sc(centers.T, centers[NFULL * W:])
    lab = jnp.pad(labels.astype(jnp.int32).reshape(NW, BPW),
                  ((0, 0), (0, BPW))).reshape(NW, 2 * KCH, CHUNK)
    zr = z.reshape(NW, KCH, CHUNK // 2, W)
    partials = _center_loss_sc(zr, lab, tab)
    return jnp.sum(partials)


# speedup vs baseline: 1.2309x; 1.2309x over previous
import functools

import jax
import jax.numpy as jnp
from jax import lax
from jax.experimental import pallas as pl
from jax.experimental.pallas import tpu as pltpu
from jax.experimental.pallas import tpu_sc as plsc

NC = 2
NS = 16
L = 16
NW = NC * NS

B = 16384
D = 64
W = 128
V = 1000000
CHUNK = 128
BPW = B // NW
KCH = BPW // CHUNK
JG = D // L

NFULL = V // W
TAIL = V - NFULL * W
ROUNDS = NFULL // NW
REM = NFULL - ROUNDS * NW

_mesh = plsc.VectorSubcoreMesh(core_axis_name="c", subcore_axis_name="s")


@functools.partial(
    pl.kernel,
    mesh=_mesh,
    out_type=jax.ShapeDtypeStruct((V, W), jnp.float32),
    compiler_params=pltpu.CompilerParams(use_tc_tiling_on_sc=True,
                                         needs_layout_passes=False),
    scratch_types=[
        pltpu.VMEM((2, D, W), jnp.float32),
        pltpu.VMEM((2, W, W), jnp.float32),
        pltpu.VMEM((TAIL, D), jnp.float32),
        pltpu.SemaphoreType.DMA,
        pltpu.SemaphoreType.DMA,
    ],
)
def _widen_sc(tab_t_hbm, tail_hbm, out_hbm, slab_v, obuf_v, tl_v, isem, osem):
    c = lax.axis_index("c")
    s = lax.axis_index("s")
    wid = c * NS + s

    f_idx = [lax.iota(jnp.int32, L) + jj * L for jj in range(JG)]

    def block_of(i):
        return wid + i * NW

    def fire_in(i, buf):
        j = block_of(i)
        return pltpu.async_copy(
            tab_t_hbm.at[:, pl.ds(pl.multiple_of(j * W, W), W)],
            slab_v.at[buf], isem)

    cvecs = [lax.iota(jnp.int32, L) + cg * L for cg in range(W // L)]

    def transpose(buf):
        def f_step(f, carry):
            fv = jnp.full((L,), f, jnp.int32)
            for cg in range(W // L):
                v = slab_v[buf, f, pl.ds(cg * L, L)]
                plsc.store_scatter(obuf_v.at[buf], [cvecs[cg], fv], v)
            return carry
        lax.fori_loop(0, D, f_step, 0, unroll=2)

    def fire_out(i, buf):
        j = block_of(i)
        return pltpu.async_copy(
            obuf_v.at[buf],
            out_hbm.at[pl.ds(pl.multiple_of(j * W, W), W)], osem)

    def wait_in(buf):
        pltpu.make_async_copy(
            tab_t_hbm.at[:, pl.ds(0, W)], slab_v.at[buf], isem).wait()

    def wait_out(buf):
        pltpu.make_async_copy(
            obuf_v.at[buf], out_hbm.at[pl.ds(0, W)], osem).wait()

    def fire_next(i):
        jn = jnp.where(i + 1 < ROUNDS,
                       wid + (i + 1) * NW,
                       ROUNDS * NW + jnp.minimum(wid, REM - 1))
        return pltpu.async_copy(
            tab_t_hbm.at[:, pl.ds(pl.multiple_of(jn * W, W), W)],
            slab_v.at[(i + 1) % 2], isem)

    fire_in(0, 0)
    def pair_step(g, carry):
        for b in range(2):
            i = g * 2 + b
            wait_in(b)
            fire_next(i)
            @pl.when(i >= 2)
            def _():
                wait_out(b)
            transpose(b)
            fire_out(i, b)
        return carry
    lax.fori_loop(0, ROUNDS // 2, pair_step, 0)
    wait_out(0)
    wait_out(1)
    wait_in(ROUNDS % 2)

    @pl.when(wid < REM)
    def _():
        buf = ROUNDS % 2
        jlast = ROUNDS * NW + wid
        transpose(buf)
        pltpu.async_copy(
            obuf_v.at[buf],
            out_hbm.at[pl.ds(pl.multiple_of(jlast * W, W), W)], osem).wait()

    @pl.when(wid == REM)
    def _():
        buf = ROUNDS % 2
        pltpu.sync_copy(tail_hbm, tl_v)

        def tail_step(cc, carry):
            ci = jnp.full((L,), cc, jnp.int32)
            for jj in range(JG):
                v = tl_v[cc, pl.ds(jj * L, L)]
                plsc.store_scatter(obuf_v.at[buf], [ci, f_idx[jj]], v)
            return carry
        lax.fori_loop(0, TAIL, tail_step, 0, unroll=4)
        pltpu.async_copy(
            obuf_v.at[buf, pl.ds(0, TAIL)],
            out_hbm.at[pl.ds(NFULL * W, TAIL)], osem).wait()


@functools.partial(
    pl.kernel,
    mesh=_mesh,
    out_type=jax.ShapeDtypeStruct((NW, L), jnp.float32),
    compiler_params=pltpu.CompilerParams(use_tc_tiling_on_sc=True),
    scratch_types=[
        pltpu.VMEM((2 * KCH, CHUNK), jnp.int32),
        pltpu.VMEM((KCH, CHUNK, W), jnp.float32),
        pltpu.VMEM((KCH, CHUNK // 2, W), jnp.float32),
        pltpu.VMEM((L,), jnp.float32),
        pltpu.SemaphoreType.DMA,
        pltpu.SemaphoreType.DMA,
    ],
)
def _center_loss_sc(z_hbm, lab_hbm, tab_hbm, out_hbm,
                    idx_v, rows_v, z_v, acc_v, sem, zsem):
    c = lax.axis_index("c")
    s = lax.axis_index("s")
    wid = c * NS + s

    pltpu.sync_copy(lab_hbm.at[wid], idx_v)
    zcp = pltpu.async_copy(z_hbm.at[wid], z_v, zsem)
    gathers = [
        pltpu.async_copy(tab_hbm.at[idx_v.at[k]], rows_v.at[k], sem)
        for k in range(KCH)
    ]
    zcp.wait()
    for cp in gathers:
        cp.wait()

    accs = (jnp.zeros((L,), jnp.float32),) * JG
    for k in range(KCH):
        def row_step(i, a, k=k):
            out = list(a)
            for p in range(2):
                for j in range(JG):
                    zv = z_v[k, i, pl.ds(p * D + j * L, L)]
                    cv = rows_v[k, 2 * i + p, pl.ds(j * L, L)]
                    d = zv - cv
                    out[j] = out[j] + d * d
            return tuple(out)
        accs = lax.fori_loop(0, CHUNK // 2, row_step, accs)

    tot = accs[0]
    for j in range(1, JG):
        tot = tot + accs[j]
    acc_v[...] = tot * (1.0 / B)
    pltpu.sync_copy(acc_v, out_hbm.at[wid])


def kernel(z, labels, centers):
    tab = _widen_sc(centers.T, centers[NFULL * W:])
    lab = jnp.pad(labels.astype(jnp.int32).reshape(NW, BPW),
                  ((0, 0), (0, BPW))).reshape(NW, 2 * KCH, CHUNK)
    zr = z.reshape(NW, KCH, CHUNK // 2, W)
    partials = _center_loss_sc(zr, lab, tab)
    return jnp.sum(partials)


# speedup vs baseline: 1.7156x; 1.3938x over previous
import functools

import jax
import jax.numpy as jnp
from jax import lax
from jax.experimental import pallas as pl
from jax.experimental.pallas import tpu as pltpu
from jax.experimental.pallas import tpu_sc as plsc

NC = 2
NS = 16
L = 16
NW = NC * NS

B = 16384
D = 64
W = 128
V = 1000000
CHUNK = 128
BPW = B // NW
KCH = BPW // CHUNK
JG = D // L

NFULL = V // W
TAIL = V - NFULL * W
ROUNDS = NFULL // NW
REM = NFULL - ROUNDS * NW

_mesh = plsc.VectorSubcoreMesh(core_axis_name="c", subcore_axis_name="s")


@functools.partial(
    pl.kernel,
    mesh=_mesh,
    out_type=jax.ShapeDtypeStruct((V, W), jnp.float32),
    compiler_params=pltpu.CompilerParams(use_tc_tiling_on_sc=True,
                                         needs_layout_passes=False),
    scratch_types=[
        pltpu.VMEM((2, D, W), jnp.float32),
        pltpu.VMEM((2, W, W), jnp.float32),
        pltpu.VMEM((TAIL, D), jnp.float32),
        pltpu.SemaphoreType.DMA,
        pltpu.SemaphoreType.DMA,
    ],
)
def _widen_sc(tab_t_hbm, tail_hbm, out_hbm, slab_v, obuf_v, tl_v, isem, osem):
    c = lax.axis_index("c")
    s = lax.axis_index("s")
    wid = c * NS + s

    f_idx = [lax.iota(jnp.int32, L) + jj * L for jj in range(JG)]

    def block_of(i):
        return wid + i * NW

    def fire_in(i, buf):
        j = block_of(i)
        return pltpu.async_copy(
            tab_t_hbm.at[:, pl.ds(pl.multiple_of(j * W, W), W)],
            slab_v.at[buf], isem)

    cvecs = [lax.iota(jnp.int32, L) + cg * L for cg in range(W // L)]

    def transpose(buf):
        @plsc.parallel_loop(0, D, unroll=4)
        def f_step(f):
            fv = jnp.full((L,), f, jnp.int32)
            for cg in range(W // L):
                v = slab_v[buf, f, pl.ds(cg * L, L)]
                plsc.store_scatter(obuf_v.at[buf], [cvecs[cg], fv], v)

    def fire_out(i, buf):
        j = block_of(i)
        return pltpu.async_copy(
            obuf_v.at[buf],
            out_hbm.at[pl.ds(pl.multiple_of(j * W, W), W)], osem)

    def wait_in(buf):
        pltpu.make_async_copy(
            tab_t_hbm.at[:, pl.ds(0, W)], slab_v.at[buf], isem).wait()

    def wait_out(buf):
        pltpu.make_async_copy(
            obuf_v.at[buf], out_hbm.at[pl.ds(0, W)], osem).wait()

    def fire_next(i):
        jn = jnp.where(i + 1 < ROUNDS,
                       wid + (i + 1) * NW,
                       ROUNDS * NW + jnp.minimum(wid, REM - 1))
        return pltpu.async_copy(
            tab_t_hbm.at[:, pl.ds(pl.multiple_of(jn * W, W), W)],
            slab_v.at[(i + 1) % 2], isem)

    fire_in(0, 0)
    def pair_step(g, carry):
        for b in range(2):
            i = g * 2 + b
            wait_in(b)
            fire_next(i)
            @pl.when(i >= 2)
            def _():
                wait_out(b)
            transpose(b)
            fire_out(i, b)
        return carry
    lax.fori_loop(0, ROUNDS // 2, pair_step, 0)
    wait_out(0)
    wait_out(1)
    wait_in(ROUNDS % 2)

    @pl.when(wid < REM)
    def _():
        buf = ROUNDS % 2
        jlast = ROUNDS * NW + wid
        transpose(buf)
        pltpu.async_copy(
            obuf_v.at[buf],
            out_hbm.at[pl.ds(pl.multiple_of(jlast * W, W), W)], osem).wait()

    @pl.when(wid == REM)
    def _():
        buf = ROUNDS % 2
        pltpu.sync_copy(tail_hbm, tl_v)

        def tail_step(cc, carry):
            ci = jnp.full((L,), cc, jnp.int32)
            for jj in range(JG):
                v = tl_v[cc, pl.ds(jj * L, L)]
                plsc.store_scatter(obuf_v.at[buf], [ci, f_idx[jj]], v)
            return carry
        lax.fori_loop(0, TAIL, tail_step, 0, unroll=4)
        pltpu.async_copy(
            obuf_v.at[buf, pl.ds(0, TAIL)],
            out_hbm.at[pl.ds(NFULL * W, TAIL)], osem).wait()


@functools.partial(
    pl.kernel,
    mesh=_mesh,
    out_type=jax.ShapeDtypeStruct((NW, L), jnp.float32),
    compiler_params=pltpu.CompilerParams(use_tc_tiling_on_sc=True),
    scratch_types=[
        pltpu.VMEM((2 * KCH, CHUNK), jnp.int32),
        pltpu.VMEM((KCH, CHUNK, W), jnp.float32),
        pltpu.VMEM((KCH, CHUNK // 2, W), jnp.float32),
        pltpu.VMEM((L,), jnp.float32),
        pltpu.SemaphoreType.DMA,
        pltpu.SemaphoreType.DMA,
    ],
)
def _center_loss_sc(z_hbm, lab_hbm, tab_hbm, out_hbm,
                    idx_v, rows_v, z_v, acc_v, sem, zsem):
    c = lax.axis_index("c")
    s = lax.axis_index("s")
    wid = c * NS + s

    pltpu.sync_copy(lab_hbm.at[wid], idx_v)
    zcp = pltpu.async_copy(z_hbm.at[wid], z_v, zsem)
    gathers = [
        pltpu.async_copy(tab_hbm.at[idx_v.at[k]], rows_v.at[k], sem)
        for k in range(KCH)
    ]
    zcp.wait()
    for cp in gathers:
        cp.wait()

    accs = (jnp.zeros((L,), jnp.float32),) * JG
    for k in range(KCH):
        def row_step(i, a, k=k):
            out = list(a)
            for p in range(2):
                for j in range(JG):
                    zv = z_v[k, i, pl.ds(p * D + j * L, L)]
                    cv = rows_v[k, 2 * i + p, pl.ds(j * L, L)]
                    d = zv - cv
                    out[j] = out[j] + d * d
            return tuple(out)
        accs = lax.fori_loop(0, CHUNK // 2, row_step, accs)

    tot = accs[0]
    for j in range(1, JG):
        tot = tot + accs[j]
    acc_v[...] = tot * (1.0 / B)
    pltpu.sync_copy(acc_v, out_hbm.at[wid])


def kernel(z, labels, centers):
    tab = _widen_sc(centers.T, centers[NFULL * W:])
    lab = jnp.pad(labels.astype(jnp.int32).reshape(NW, BPW),
                  ((0, 0), (0, BPW))).reshape(NW, 2 * KCH, CHUNK)
    zr = z.reshape(NW, KCH, CHUNK // 2, W)
    partials = _center_loss_sc(zr, lab, tab)
    return jnp.sum(partials)


# speedup vs baseline: 2.7140x; 1.5820x over previous
import functools

import jax
import jax.numpy as jnp
from jax import lax
from jax.experimental import pallas as pl
from jax.experimental.pallas import tpu as pltpu
from jax.experimental.pallas import tpu_sc as plsc

NC = 2
NS = 16
L = 16
NW = NC * NS

B = 16384
D = 64
W = 128
CHUNK = 128
BPW = B // NW
KCH = BPW // CHUNK
JG = D // L

_mesh = plsc.VectorSubcoreMesh(core_axis_name="c", subcore_axis_name="s")


@functools.partial(
    pl.kernel,
    mesh=_mesh,
    out_type=jax.ShapeDtypeStruct((NW, L), jnp.float32),
    compiler_params=pltpu.CompilerParams(use_tc_tiling_on_sc=True),
    scratch_types=[
        pltpu.VMEM((2 * KCH, CHUNK), jnp.int32),
        pltpu.VMEM((KCH, CHUNK, W), jnp.float32),
        pltpu.VMEM((KCH, CHUNK // 2, W), jnp.float32),
        pltpu.VMEM((L,), jnp.float32),
        pltpu.SemaphoreType.DMA,
        pltpu.SemaphoreType.DMA,
    ],
)
def _center_loss_sc(z_hbm, lab_hbm, tab_hbm, out_hbm,
                    idx_v, rows_v, z_v, acc_v, sem, zsem):
    c = lax.axis_index("c")
    s = lax.axis_index("s")
    wid = c * NS + s

    pltpu.sync_copy(lab_hbm.at[wid], idx_v)
    zcp = pltpu.async_copy(z_hbm.at[wid], z_v, zsem)
    gathers = [
        pltpu.async_copy(tab_hbm.at[idx_v.at[k]], rows_v.at[k], sem)
        for k in range(KCH)
    ]
    zcp.wait()
    for cp in gathers:
        cp.wait()

    accs = (jnp.zeros((L,), jnp.float32),) * JG
    for k in range(KCH):
        def row_step(i, a, k=k):
            out = list(a)
            for p in range(2):
                for j in range(JG):
                    zv = z_v[k, i, pl.ds(p * D + j * L, L)]
                    cv = rows_v[k, 2 * i + p, pl.ds(j * L, L)]
                    d = zv - cv
                    out[j] = out[j] + d * d
            return tuple(out)
        accs = lax.fori_loop(0, CHUNK // 2, row_step, accs)

    tot = accs[0]
    for j in range(1, JG):
        tot = tot + accs[j]
    acc_v[...] = tot * (1.0 / B)
    pltpu.sync_copy(acc_v, out_hbm.at[wid])


def kernel(z, labels, centers):
    tab = jnp.pad(centers, ((0, 0), (0, W - D)))
    lab = jnp.pad(labels.astype(jnp.int32).reshape(NW, BPW),
                  ((0, 0), (0, BPW))).reshape(NW, 2 * KCH, CHUNK)
    zr = z.reshape(NW, KCH, CHUNK // 2, W)
    partials = _center_loss_sc(zr, lab, tab)
    return jnp.sum(partials)
